# baseline (speedup 1.0000x reference)
import jax
import jax.numpy as jnp
from jax.experimental import pallas as pl

N_HID = 32
N_HEAD = 4
NUM_GRAPHS = 1250


def _masked_bn(x, mask, g, b):
    m = mask[..., None].astype(x.dtype)
    axes = tuple(range(x.ndim - 1))
    cnt = jnp.maximum(m.sum(axis=axes), 1.0)
    mean = (x * m).sum(axis=axes) / cnt
    var = (((x - mean) ** 2) * m).sum(axis=axes) / cnt
    return ((x - mean) / jnp.sqrt(var + 1e-5)) * g + b


def _layernorm(x, g, b):
    mu = x.mean(axis=-1, keepdims=True)
    var = x.var(axis=-1, keepdims=True)
    return (x - mu) / jnp.sqrt(var + 1e-5) * g + b


def _gin_conv(x, src, dst, edge_attr, p):
    e = edge_attr @ p['We'] + p['be']
    msg = jax.nn.relu(x[:, src, :] + e[None, :, :])
    agg = jnp.zeros_like(x).at[:, dst, :].add(msg)
    h = (1.0 + p['eps']) * x + agg
    return jax.nn.relu(h @ p['W1'] + p['b1']) @ p['W2'] + p['b2']


def _phi(x, edge_index, edge_attr, mask_full, layers):
    x = jnp.transpose(x, (1, 0, 2))
    m = jnp.transpose(mask_full, (1, 0))
    src, dst = edge_index[0], edge_index[1]
    prev = 0.0
    for p in layers:
        x = _gin_conv(x, src, dst, edge_attr, p)
        x = jnp.where(m[..., None], x, 0.0)
        x = _masked_bn(x, m, p['bng'], p['bnb'])
        x = jax.nn.relu(x)
        x = x + prev
        prev = x
    return jnp.transpose(x, (1, 0, 2))


def _enc(x, mask, p):
    h = x @ p['W1'] + p['b1']
    h = jax.nn.relu(_masked_bn(h, mask, p['g1'], p['be1']))
    h = h @ p['W2'] + p['b2']
    h = jax.nn.relu(_masked_bn(h, mask, p['g2'], p['be2']))
    return h


def _attn_layer(x, mask, p):
    n, k, d = x.shape
    dh = d // N_HEAD
    q = (x @ p['Wq'] + p['bq']).reshape(n, k, N_HEAD, dh)
    kk = (x @ p['Wk'] + p['bk']).reshape(n, k, N_HEAD, dh)
    v = (x @ p['Wv'] + p['bv']).reshape(n, k, N_HEAD, dh)
    s = jnp.einsum('nqhd,nkhd->nhqk', q, kk) / jnp.sqrt(jnp.asarray(dh, jnp.float32))
    s = jnp.where(mask[:, None, None, :], s, -1e9)
    a = jax.nn.softmax(s, axis=-1)
    o = jnp.einsum('nhqk,nkhd->nqhd', a, v).reshape(n, k, d) @ p['Wo'] + p['bo']
    x = _layernorm(x + o, p['ln1g'], p['ln1b'])
    f = jax.nn.relu(x @ p['F1'] + p['f1b']) @ p['F2'] + p['f2b']
    return _layernorm(x + f, p['ln2g'], p['ln2b'])


def _rho(x, pos, mask, layers, out_p):
    x = x + pos
    for p in layers:
        x = _attn_layer(x, mask, p)
    x = x.sum(axis=1)
    h = x @ out_p['W']
    mu = h.mean(axis=0)
    var = h.var(axis=0)
    return (h - mu) / jnp.sqrt(var + 1e-5) * out_p['g'] + out_p['b']


def _identity_block(x_ref, o_ref):
    o_ref[...] = x_ref[...]


def kernel(eigen_vectors, eigen_values, edge_attr, edge_index, batch, params):
    n, k = eigen_vectors.shape
    size = jax.ops.segment_sum(jnp.ones((n,), jnp.int32), batch, num_segments=NUM_GRAPHS)
    mask = (jnp.arange(k)[None, :] < size[:, None])[batch]
    x = eigen_vectors[..., None]
    pos = _enc(eigen_values[..., None], mask, params['enc'])
    h = _phi(x, edge_index, edge_attr, mask, params['phi']) + _phi(-x, edge_index, edge_attr, mask, params['phi'])
    out = _rho(h, pos, mask, params['rho'], params['out'])
    out = pl.pallas_call(
        _identity_block,
        out_shape=jax.ShapeDtypeStruct(out.shape, out.dtype),
    )(out)
    return out


# speedup vs baseline: 12.3337x; 12.3337x over previous
import functools

import jax
import jax.numpy as jnp
import numpy as np
from jax import lax
from jax.experimental import pallas as pl
from jax.experimental.pallas import tpu as pltpu
from jax.experimental.pallas import tpu_sc as plsc

N = 10000
E = 160000
K = 8
C = 16
H = 32
NB = 1000
GRID_N = N // NB
EB = 4000
GRID_E = E // EB
NC, NS = 2, 16
EPT = E // (NC * NS)
BATCH = 40
NBATCH = EPT // BATCH
NPT = 624
EPS_BN = 1e-5
F32 = jnp.float32
HI = lax.Precision.HIGHEST


def _sc_gin_body(n_chunks, cw, ew, x_hbm, e_hbm, src_hbm, dst_hbm, zeros_hbm,
                 out_hbm, idx_src, idx_dst, rows_v, e_v, sem, agg_sh):
    c = lax.axis_index("c")
    s = lax.axis_index("s")
    base_t = (c * NS + s) * EPT
    qr = cw // 16
    qe = ew // 16
    tail = NS * NPT
    for j in range(n_chunks):
        pltpu.sync_copy(zeros_hbm, agg_sh.at[pl.ds(s * NPT, NPT)])

        @pl.when(s == NS - 1)
        def _():
            pltpu.sync_copy(zeros_hbm.at[pl.ds(0, N - tail)],
                            agg_sh.at[pl.ds(tail, N - tail)])

        plsc.subcore_barrier()

        def batch_body(b, carry, j=j):
            base = base_t + b * BATCH
            pltpu.sync_copy(src_hbm.at[pl.ds(base, BATCH)], idx_src)
            pltpu.sync_copy(dst_hbm.at[pl.ds(base, BATCH)], idx_dst)
            pltpu.sync_copy(e_hbm.at[pl.ds(base, BATCH)], e_v)
            pltpu.async_copy(x_hbm.at[j].at[idx_src], rows_v, sem).wait()

            def edge_body(i, _):
                for q in range(qr):
                    ev = e_v[i, pl.ds((q % qe) * 16, 16)]
                    rv = rows_v[i, pl.ds(q * 16, 16)]
                    rows_v[i, pl.ds(q * 16, 16)] = jnp.maximum(rv + ev, 0.0)
                return 0

            lax.fori_loop(0, BATCH, edge_body, 0, unroll=2)
            pltpu.sync_copy(rows_v, agg_sh.at[idx_dst], add=True)
            return carry

        lax.fori_loop(0, NBATCH, batch_body, 0)
        plsc.subcore_barrier()
        pltpu.sync_copy(agg_sh.at[pl.ds(s * NPT, NPT)],
                        out_hbm.at[c].at[j].at[pl.ds(s * NPT, NPT)])

        @pl.when(s == NS - 1)
        def _():
            pltpu.sync_copy(agg_sh.at[pl.ds(tail, N - tail)],
                            out_hbm.at[c].at[j].at[pl.ds(tail, N - tail)])

        plsc.subcore_barrier()


def _sc_gin(n_chunks, cw, ew, x_tab, e_tab, src, dst, zeros):
    mesh = plsc.VectorSubcoreMesh(core_axis_name="c", subcore_axis_name="s")
    body = functools.partial(_sc_gin_body, n_chunks, cw, ew)
    return pl.kernel(
        body,
        out_type=jax.ShapeDtypeStruct((NC, n_chunks, N, cw), F32),
        mesh=mesh,
        scratch_types=[
            pltpu.VMEM((BATCH,), jnp.int32),
            pltpu.VMEM((BATCH,), jnp.int32),
            pltpu.VMEM((BATCH, cw), F32),
            pltpu.VMEM((BATCH, ew), F32),
            pltpu.SemaphoreType.DMA,
            pltpu.VMEM_SHARED((N, cw), F32),
        ],
    )(x_tab, e_tab, src, dst, zeros)


def _prep_nodes_body(batch_ref, ev_ref, maskf_ref, x0_ref, cnt_ref):
    b = batch_ref[...].reshape(N, 1)
    tot = jnp.zeros((N, 1), jnp.int32)
    for j in range(1, K):
        up = jnp.concatenate([jnp.full((j, 1), -1, jnp.int32), b[:N - j]], axis=0)
        dn = jnp.concatenate([b[j:], jnp.full((j, 1), -2, jnp.int32)], axis=0)
        tot = tot + (up == b).astype(jnp.int32) + (dn == b).astype(jnp.int32)
    kio = lax.broadcasted_iota(jnp.int32, (N, K), 1)
    maskf = (kio <= tot).astype(F32)
    maskf_ref[...] = maskf
    ev = ev_ref[...]
    x0_ref[...] = jnp.concatenate(
        [ev, -ev, jnp.zeros((N, 4 * H - C), F32)], axis=1).reshape(1, N, 4 * H)
    cnt_ref[...] = jnp.maximum(jnp.sum(maskf), 1.0).reshape(1, 1)


def _prep_nodes(batch, eigen_vectors):
    return pl.pallas_call(
        _prep_nodes_body,
        out_shape=(
            jax.ShapeDtypeStruct((N, K), F32),
            jax.ShapeDtypeStruct((1, N, 4 * H), F32),
            jax.ShapeDtypeStruct((1, 1), F32),
        ),
    )(batch, eigen_vectors)


def _prep_edges_body(ea_ref, we1_ref, be1_ref, we2_ref, be2_ref, we3_ref,
                     be3_ref, e1_ref, e2_ref, e3_ref):
    ea = ea_ref[...]
    e1 = jnp.dot(ea, we1_ref[...], preferred_element_type=F32) + be1_ref[...]
    e1_ref[...] = jnp.broadcast_to(e1, (EB, C))
    e2_ref[...] = jnp.dot(ea, we2_ref[...], preferred_element_type=F32) + be2_ref[...]
    e3_ref[...] = jnp.dot(ea, we3_ref[...], preferred_element_type=F32) + be3_ref[...]


def _prep_edges(edge_attr, we1, be1, we2, be2, we3, be3):
    wspec = lambda a: pl.BlockSpec(a.shape, lambda i: (0,) * a.ndim)
    return pl.pallas_call(
        _prep_edges_body,
        grid=(GRID_E,),
        in_specs=[pl.BlockSpec((EB, 4), lambda i: (i, 0)),
                  wspec(we1), wspec(be1), wspec(we2), wspec(be2),
                  wspec(we3), wspec(be3)],
        out_specs=(pl.BlockSpec((EB, C), lambda i: (i, 0)),
                   pl.BlockSpec((EB, H), lambda i: (i, 0)),
                   pl.BlockSpec((EB, H), lambda i: (i, 0))),
        out_shape=(jax.ShapeDtypeStruct((E, C), F32),
                   jax.ShapeDtypeStruct((E, H), F32),
                   jax.ShapeDtypeStruct((E, H), F32)),
    )(edge_attr, we1, be1, we2, be2, we3, be3)


def _fold_mat(groups, width):
    f = np.zeros((groups * width, width), np.float32)
    for g in range(groups):
        f[g * width:(g + 1) * width] = np.eye(width, dtype=np.float32)
    return jnp.asarray(f)


def _acc_stats(i, stats_ref, ssum, ssq):
    new = jnp.concatenate([ssum, ssq], axis=0)

    @pl.when(i == 0)
    def _():
        stats_ref[...] = new

    @pl.when(i != 0)
    def _():
        stats_ref[...] = stats_ref[...] + new


def _enc1_body(ev_ref, maskf_ref, w1t_ref, b1t_ref, fold8_ref, h1_ref, st_ref):
    i = pl.program_id(0)
    ev = ev_ref[...]
    m256 = jnp.broadcast_to(maskf_ref[...][:, :, None], (NB, K, H)).reshape(NB, K * H)
    rep = jnp.broadcast_to(ev[:, :, None], (NB, K, H)).reshape(NB, K * H)
    h1 = rep * w1t_ref[...] + b1t_ref[...]
    h1_ref[...] = h1
    hm = h1 * m256
    fold8 = fold8_ref[...]
    ssum = jnp.sum(jnp.dot(hm, fold8, preferred_element_type=F32, precision=HI), axis=0, keepdims=True)
    ssq = jnp.sum(jnp.dot(hm * h1, fold8, preferred_element_type=F32, precision=HI), axis=0, keepdims=True)
    _acc_stats(i, st_ref, ssum, ssq)


def _enc2_body(h1_ref, maskf_ref, st1_ref, cnt_ref, g1t_ref, be1t_ref,
               w2bd_ref, b2t_ref, fold8_ref, h2_ref, st_ref):
    i = pl.program_id(0)
    cnt = cnt_ref[0, 0]
    mu = st1_ref[0:1] / cnt
    var = st1_ref[1:2] / cnt - mu * mu
    rstd = lax.rsqrt(var + EPS_BN)
    mu_t = jnp.tile(mu, (1, K))
    rstd_t = jnp.tile(rstd, (1, K))
    hb = jnp.maximum((h1_ref[...] - mu_t) * rstd_t * g1t_ref[...] + be1t_ref[...], 0.0)
    h2 = jnp.dot(hb, w2bd_ref[...], preferred_element_type=F32) + b2t_ref[...]
    h2_ref[...] = h2
    m256 = jnp.broadcast_to(maskf_ref[...][:, :, None], (NB, K, H)).reshape(NB, K * H)
    hm = h2 * m256
    fold8 = fold8_ref[...]
    ssum = jnp.sum(jnp.dot(hm, fold8, preferred_element_type=F32, precision=HI), axis=0, keepdims=True)
    ssq = jnp.sum(jnp.dot(hm * h2, fold8, preferred_element_type=F32, precision=HI), axis=0, keepdims=True)
    _acc_stats(i, st_ref, ssum, ssq)


def _gin_a_body(layer1, eps_ref, xp_ref, agg_ref, maskf_ref, w1_ref, b1t_ref,
                w2bd_ref, b2t_ref, fold4_ref, y_ref, st_ref):
    i = pl.program_id(0)
    eps1 = 1.0 + eps_ref[0, 0]
    maskf = maskf_ref[...]
    fold4 = fold4_ref[...]
    sums = [None, None]
    sqs = [None, None]
    if layer1:
        h = (eps1 * xp_ref[...][0, :, :C] + agg_ref[...][0, 0, :, :C]
             + agg_ref[...][1, 0, :, :C])
    for j in range(4):
        if layer1:
            hj = h[:, 4 * j:4 * j + 4]
            hb = jnp.broadcast_to(hj[:, :, None], (NB, 4, H)).reshape(NB, 4 * H)
            t = jnp.maximum(hb * w1_ref[...] + b1t_ref[...], 0.0)
        else:
            hj = (eps1 * xp_ref[...][j] + agg_ref[...][0, j] + agg_ref[...][1, j])
            t = jnp.maximum(jnp.dot(hj, w1_ref[...], preferred_element_type=F32) + b1t_ref[...], 0.0)
        yj = jnp.dot(t, w2bd_ref[...], preferred_element_type=F32) + b2t_ref[...]
        mj = maskf[:, 4 * (j % 2):4 * (j % 2) + 4]
        yj = yj * jnp.broadcast_to(mj[:, :, None], (NB, 4, H)).reshape(NB, 4 * H)
        y_ref[j, :, :] = yj
        g = j // 2
        sj = jnp.sum(jnp.dot(yj, fold4, preferred_element_type=F32, precision=HI), axis=0, keepdims=True)
        qj = jnp.sum(jnp.dot(yj * yj, fold4, preferred_element_type=F32, precision=HI), axis=0, keepdims=True)
        sums[g] = sj if sums[g] is None else sums[g] + sj
        sqs[g] = qj if sqs[g] is None else sqs[g] + qj
    _acc_stats(i, st_ref, jnp.concatenate(sums, axis=0), jnp.concatenate(sqs, axis=0))


def _gin_b_body(has_prev, y_ref, st_ref, cnt_ref, bngt_ref, bnbt_ref, *rest):
    if has_prev:
        prev_ref, x_ref = rest
    else:
        (x_ref,) = rest
    cnt = cnt_ref[0, 0]
    for j in range(4):
        g = j // 2
        mu = st_ref[g:g + 1] / cnt
        var = st_ref[2 + g:3 + g] / cnt - mu * mu
        rstd = lax.rsqrt(var + EPS_BN)
        mu_t = jnp.tile(mu, (1, 4))
        rstd_t = jnp.tile(rstd, (1, 4))
        xj = jnp.maximum((y_ref[j, :, :] - mu_t) * rstd_t * bngt_ref[...] + bnbt_ref[...], 0.0)
        if has_prev:
            xj = xj + prev_ref[j, :, :]
        x_ref[j, :, :] = xj


def _gin_a(layer1, eps, x_prev, agg, maskf, w1, b1t, w2bd, b2t):
    n_chunks = 1 if layer1 else 4
    cw = 4 * H
    fold4 = _fold_mat(4, H)
    wspec = lambda a: pl.BlockSpec(a.shape, lambda i: (0,) * a.ndim)
    return pl.pallas_call(
        functools.partial(_gin_a_body, layer1),
        grid=(GRID_N,),
        in_specs=[wspec(eps),
                  pl.BlockSpec((n_chunks, NB, cw), lambda i: (0, i, 0)),
                  pl.BlockSpec((NC, n_chunks, NB, cw), lambda i: (0, 0, i, 0)),
                  pl.BlockSpec((NB, K), lambda i: (i, 0)),
                  wspec(w1), wspec(b1t), wspec(w2bd), wspec(b2t), wspec(fold4)],
        out_specs=(pl.BlockSpec((4, NB, 4 * H), lambda i: (0, i, 0)),
                   pl.BlockSpec((4, H), lambda i: (0, 0))),
        out_shape=(jax.ShapeDtypeStruct((4, N, 4 * H), F32),
                   jax.ShapeDtypeStruct((4, H), F32)),
    )(eps, x_prev, agg, maskf, w1, b1t, w2bd, b2t, fold4)


def _gin_b(y, stats, cnt, bngt, bnbt, prev):
    has_prev = prev is not None
    wspec = lambda a: pl.BlockSpec(a.shape, lambda i: (0,) * a.ndim)
    in_specs = [pl.BlockSpec((4, NB, 4 * H), lambda i: (0, i, 0)),
                wspec(stats), wspec(cnt), wspec(bngt), wspec(bnbt)]
    args = [y, stats, cnt, bngt, bnbt]
    if has_prev:
        in_specs.append(pl.BlockSpec((4, NB, 4 * H), lambda i: (0, i, 0)))
        args.append(prev)
    return pl.pallas_call(
        functools.partial(_gin_b_body, has_prev),
        grid=(GRID_N,),
        in_specs=in_specs,
        out_specs=pl.BlockSpec((4, NB, 4 * H), lambda i: (0, i, 0)),
        out_shape=jax.ShapeDtypeStruct((4, N, 4 * H), F32),
    )(*args)


def _rho_a_body(x3_ref, h2_ref, maskf_ref, st2_ref, cnt_ref, consts_ref,
                g2t_ref, be2t_ref, wq_ref, bq_ref, wk_ref, bk_ref, wv_ref,
                bv_ref, wo_ref, bo_ref, f1_ref, f1b_ref, f2_ref, f2b_ref,
                ln1g_ref, ln1b_ref, ln2g_ref, ln2b_ref, wout_ref,
                g_ref, st_ref):
    i = pl.program_id(0)
    KH = K * H
    gsum = consts_ref[...][0]
    gexp = consts_ref[...][1]
    ggrp = consts_ref[...][2]
    ggrpt = consts_ref[...][3]
    ssum = consts_ref[...][4]

    cnt = cnt_ref[0, 0]
    mu = st2_ref[0:1] / cnt
    var = st2_ref[1:2] / cnt - mu * mu
    rstd = lax.rsqrt(var + EPS_BN)
    pos = jnp.maximum((h2_ref[...] - jnp.tile(mu, (1, K))) * jnp.tile(rstd, (1, K))
                      * g2t_ref[...] + be2t_ref[...], 0.0)

    x3 = x3_ref[...]
    xs = []
    for k in range(K):
        j, o = k // 4, (k % 4) * H
        xs.append(x3[j, :, o:o + H] + x3[2 + j, :, o:o + H])
    x = jnp.concatenate(xs, axis=1) + pos

    maskf = maskf_ref[...]
    neg = jnp.float32(-1e9)
    scale = jnp.float32(1.0 / np.sqrt(H // 4))

    def layernorm(z, gt, bt):
        m8 = jnp.dot(z, ggrp, preferred_element_type=F32, precision=HI)[:, :K] * (1.0 / H)
        v8 = jnp.dot(z * z, ggrp, preferred_element_type=F32, precision=HI)[:, :K] * (1.0 / H) - m8 * m8
        r8 = lax.rsqrt(v8 + EPS_BN)
        pad = jnp.zeros((NB, KH - K), F32)
        mz = jnp.dot(jnp.concatenate([m8, pad], axis=1), ggrpt, preferred_element_type=F32, precision=HI)
        rz = jnp.dot(jnp.concatenate([r8, pad], axis=1), ggrpt, preferred_element_type=F32, precision=HI)
        return (z - mz) * rz * gt + bt

    for l in range(2):
        q = jnp.dot(x, wq_ref[...][l], preferred_element_type=F32) + bq_ref[...][l:l + 1]
        kk = jnp.dot(x, wk_ref[...][l], preferred_element_type=F32) + bk_ref[...][l:l + 1]
        v = jnp.dot(x, wv_ref[...][l], preferred_element_type=F32) + bv_ref[...][l:l + 1]
        s_list = []
        for kt in range(K):
            ktile = jnp.tile(kk[:, kt * H:(kt + 1) * H], (1, K))
            s = jnp.dot(q * ktile, gsum, preferred_element_type=F32, precision=HI)[:, :H] * scale
            s = jnp.where(maskf[:, kt:kt + 1] > 0.0, s, neg)
            s_list.append(s)
        m = s_list[0]
        for kt in range(1, K):
            m = jnp.maximum(m, s_list[kt])
        es = [jnp.exp(s - m) for s in s_list]
        den = es[0]
        for kt in range(1, K):
            den = den + es[kt]
        rden = 1.0 / den
        o = None
        for kt in range(K):
            a = es[kt] * rden
            apad = jnp.concatenate([a, jnp.zeros((NB, KH - H), F32)], axis=1)
            ax = jnp.dot(apad, gexp, preferred_element_type=F32, precision=HI)
            contrib = ax * jnp.tile(v[:, kt * H:(kt + 1) * H], (1, K))
            o = contrib if o is None else o + contrib
        o = jnp.dot(o, wo_ref[...][l], preferred_element_type=F32) + bo_ref[...][l:l + 1]
        x = layernorm(x + o, ln1g_ref[...][l:l + 1], ln1b_ref[...][l:l + 1])
        f = jnp.maximum(jnp.dot(x, f1_ref[...][l], preferred_element_type=F32)
                        + f1b_ref[...][l:l + 1], 0.0)
        f = jnp.dot(f, f2_ref[...][l], preferred_element_type=F32) + f2b_ref[...][l:l + 1]
        x = layernorm(x + f, ln2g_ref[...][l:l + 1], ln2b_ref[...][l:l + 1])

    xk = jnp.dot(x, ssum, preferred_element_type=F32, precision=HI)[:, :H]
    g = jnp.dot(xk, wout_ref[...], preferred_element_type=F32)
    g_ref[...] = g
    s1 = jnp.sum(g, axis=0, keepdims=True)
    s2 = jnp.sum(g * g, axis=0, keepdims=True)
    _acc_stats(i, st_ref, s1, s2)


def _rho_b_body(g_ref, st_ref, gam_ref, bet_ref, out_ref):
    mu = st_ref[0:1] / N
    var = st_ref[1:2] / N - mu * mu
    rstd = lax.rsqrt(var + EPS_BN)
    out_ref[...] = (g_ref[...] - mu) * rstd * gam_ref[...] + bet_ref[...]


def _np_consts():
    KH = K * H
    dh = H // 4
    gsum = np.zeros((KH, KH), np.float32)
    gexp = np.zeros((KH, KH), np.float32)
    for qb in range(K):
        for h in range(4):
            for d in range(dh):
                gsum[qb * H + h * dh + d, qb * 4 + h] = 1.0
                gexp[qb * 4 + h, qb * H + h * dh + d] = 1.0
    ggrp = np.zeros((KH, KH), np.float32)
    ggrpt = np.zeros((KH, KH), np.float32)
    for kq in range(K):
        for d in range(H):
            ggrp[kq * H + d, kq] = 1.0
            ggrpt[kq, kq * H + d] = 1.0
    ssum = np.zeros((KH, KH), np.float32)
    for kq in range(K):
        for d in range(H):
            ssum[kq * H + d, d] = 1.0
    return jnp.asarray(np.stack([gsum, gexp, ggrp, ggrpt, ssum]))


def kernel(eigen_vectors, eigen_values, edge_attr, edge_index, batch, params):
    pe = params['enc']
    pp = params['phi']
    pr = params['rho']
    po = params['out']
    eye8 = np.eye(K, dtype=np.float32)
    eye4 = np.eye(4, dtype=np.float32)
    kron8 = lambda w: jnp.kron(jnp.asarray(eye8), w)
    kron4 = lambda w: jnp.kron(jnp.asarray(eye4), w)
    tile8 = lambda v: jnp.tile(v.reshape(1, -1), (1, K))
    tile4 = lambda v: jnp.tile(v.reshape(1, -1), (1, 4))

    src = edge_index[0]
    dst = edge_index[1]

    maskf, x0, cnt = _prep_nodes(batch, eigen_vectors)
    e1x, e2, e3 = _prep_edges(
        edge_attr,
        pp[0]['We'], pp[0]['be'].reshape(1, 1),
        pp[1]['We'], pp[1]['be'].reshape(1, H),
        pp[2]['We'], pp[2]['be'].reshape(1, H))

    zeros128 = jnp.zeros((NPT, 4 * H), F32)

    agg1 = _sc_gin(1, 4 * H, C, x0, e1x, src, dst, zeros128)

    st1_shape = jax.ShapeDtypeStruct((2, H), F32)
    wspec = lambda a: pl.BlockSpec(a.shape, lambda i: (0,) * a.ndim)
    fold8 = _fold_mat(K, H)
    w1t = tile8(pe['W1'].reshape(H))
    h1, st1 = pl.pallas_call(
        _enc1_body,
        grid=(GRID_N,),
        in_specs=[pl.BlockSpec((NB, K), lambda i: (i, 0)),
                  pl.BlockSpec((NB, K), lambda i: (i, 0)),
                  wspec(w1t), wspec(w1t), wspec(fold8)],
        out_specs=(pl.BlockSpec((NB, K * H), lambda i: (i, 0)),
                   pl.BlockSpec((2, H), lambda i: (0, 0))),
        out_shape=(jax.ShapeDtypeStruct((N, K * H), F32), st1_shape),
    )(eigen_values, maskf, w1t, tile8(pe['b1']), fold8)

    h2, st2 = pl.pallas_call(
        _enc2_body,
        grid=(GRID_N,),
        in_specs=[pl.BlockSpec((NB, K * H), lambda i: (i, 0)),
                  pl.BlockSpec((NB, K), lambda i: (i, 0)),
                  wspec(st1), wspec(cnt), wspec(w1t), wspec(w1t),
                  pl.BlockSpec((K * H, K * H), lambda i: (0, 0)), wspec(w1t),
                  wspec(fold8)],
        out_specs=(pl.BlockSpec((NB, K * H), lambda i: (i, 0)),
                   pl.BlockSpec((2, H), lambda i: (0, 0))),
        out_shape=(jax.ShapeDtypeStruct((N, K * H), F32), st1_shape),
    )(h1, maskf, st1, cnt, tile8(pe['g1']), tile8(pe['be1']),
      kron8(pe['W2']), tile8(pe['b2']), fold8)

    x_tabs = []
    x_prev_tab = x0
    for li in range(3):
        p = pp[li]
        layer1 = li == 0
        if layer1:
            agg = agg1
        else:
            e_tab = e2 if li == 1 else e3
            agg = _sc_gin(4, 4 * H, H, x_prev_tab, e_tab, src, dst, zeros128)
        w1 = tile4(p['W1'].reshape(1, H)) if layer1 else kron4(p['W1'])
        y, st = _gin_a(layer1, p['eps'].reshape(1, 1), x_prev_tab, agg, maskf,
                       w1, tile4(p['b1']), kron4(p['W2']), tile4(p['b2']))
        prev = x_tabs[-1] if x_tabs else None
        x_tab = _gin_b(y, st, cnt, tile4(p['bng']), tile4(p['bnb']), prev)
        x_tabs.append(x_tab)
        x_prev_tab = x_tab

    consts = _np_consts()
    KH = K * H
    stk = lambda key: jnp.stack([kron8(pr[l][key]) for l in range(2)])
    stb = lambda key: jnp.stack([tile8(pr[l][key])[0] for l in range(2)])
    g_arr, stg = pl.pallas_call(
        _rho_a_body,
        grid=(GRID_N,),
        in_specs=[pl.BlockSpec((4, NB, 4 * H), lambda i: (0, i, 0)),
                  pl.BlockSpec((NB, KH), lambda i: (i, 0)),
                  pl.BlockSpec((NB, K), lambda i: (i, 0)),
                  wspec(st2), wspec(cnt),
                  pl.BlockSpec((5, KH, KH), lambda i: (0, 0, 0)),
                  wspec(w1t), wspec(w1t),
                  pl.BlockSpec((2, KH, KH), lambda i: (0, 0, 0)),
                  pl.BlockSpec((2, KH), lambda i: (0, 0)),
                  pl.BlockSpec((2, KH, KH), lambda i: (0, 0, 0)),
                  pl.BlockSpec((2, KH), lambda i: (0, 0)),
                  pl.BlockSpec((2, KH, KH), lambda i: (0, 0, 0)),
                  pl.BlockSpec((2, KH), lambda i: (0, 0)),
                  pl.BlockSpec((2, KH, KH), lambda i: (0, 0, 0)),
                  pl.BlockSpec((2, KH), lambda i: (0, 0)),
                  pl.BlockSpec((2, KH, KH), lambda i: (0, 0, 0)),
                  pl.BlockSpec((2, KH), lambda i: (0, 0)),
                  pl.BlockSpec((2, KH, KH), lambda i: (0, 0, 0)),
                  pl.BlockSpec((2, KH), lambda i: (0, 0)),
                  pl.BlockSpec((2, KH), lambda i: (0, 0)),
                  pl.BlockSpec((2, KH), lambda i: (0, 0)),
                  pl.BlockSpec((2, KH), lambda i: (0, 0)),
                  pl.BlockSpec((2, KH), lambda i: (0, 0)),
                  wspec(po['W'])],
        out_specs=(pl.BlockSpec((NB, H), lambda i: (i, 0)),
                   pl.BlockSpec((2, H), lambda i: (0, 0))),
        out_shape=(jax.ShapeDtypeStruct((N, H), F32), st1_shape),
    )(x_tabs[2], h2, maskf, st2, cnt, consts,
      tile8(pe['g2']), tile8(pe['be2']),
      stk('Wq'), stb('bq'), stk('Wk'), stb('bk'), stk('Wv'), stb('bv'),
      stk('Wo'), stb('bo'), stk('F1'), stb('f1b'), stk('F2'), stb('f2b'),
      stb('ln1g'), stb('ln1b'), stb('ln2g'), stb('ln2b'), po['W'])

    out = pl.pallas_call(
        _rho_b_body,
        in_specs=[pl.BlockSpec((N, H), lambda: (0, 0)),
                  pl.BlockSpec((2, H), lambda: (0, 0)),
                  pl.BlockSpec((1, H), lambda: (0, 0)),
                  pl.BlockSpec((1, H), lambda: (0, 0))],
        out_specs=pl.BlockSpec((N, H), lambda: (0, 0)),
        out_shape=jax.ShapeDtypeStruct((N, H), F32),
    )(g_arr, stg, po['g'].reshape(1, H), po['b'].reshape(1, H))
    return out
